# baseline (speedup 1.0000x reference)
import jax
import jax.numpy as jnp
from jax import lax
from jax.experimental import pallas as pl
from jax.experimental.pallas import tpu as pltpu, tpu_sc as plsc

D_MODEL = 64
NUM_CORES = 2
NUM_SUBCORES = 16
NUM_WORKERS = NUM_CORES * NUM_SUBCORES
LANES = 16

B_TOTAL = 1024 * 200
B_PER_W = B_TOTAL // NUM_WORKERS
CHUNK = 640
N_CHUNKS = B_PER_W // CHUNK

_SCALE = 8.0


def _emb_kernel(x_hbm, w_hbm, out_hbm, idx_v, rows_v, gsem):
    wid = lax.axis_index("s") * NUM_CORES + lax.axis_index("c")
    base = wid * B_PER_W

    for c in range(N_CHUNKS):
        off = base + c * CHUNK
        pltpu.sync_copy(x_hbm.at[pl.ds(off, CHUNK)], idx_v)
        pltpu.async_copy(w_hbm.at[idx_v], rows_v, gsem).wait()

        def scale_row(i, carry):
            for j in range(D_MODEL // LANES):
                sl = pl.ds(j * LANES, LANES)
                rows_v[i, sl] = rows_v[i, sl] * _SCALE
            return carry

        lax.fori_loop(0, CHUNK, scale_row, 0, unroll=2)
        pltpu.sync_copy(rows_v, out_hbm.at[pl.ds(off, CHUNK)])


@jax.jit
def _emb(x_flat, w):
    mesh = plsc.VectorSubcoreMesh(core_axis_name="c", subcore_axis_name="s")
    run = pl.kernel(
        _emb_kernel,
        out_type=jax.ShapeDtypeStruct((B_TOTAL, D_MODEL), jnp.float32),
        mesh=mesh,
        scratch_types=[
            pltpu.VMEM((CHUNK,), jnp.int32),
            pltpu.VMEM((CHUNK, D_MODEL), jnp.float32),
            pltpu.SemaphoreType.DMA,
        ],
        compiler_params=pltpu.CompilerParams(use_tc_tiling_on_sc=False),
    )
    return run(x_flat, w)


def kernel(x, W):
    x_flat = x.reshape(-1).astype(jnp.int32)
    out = _emb(x_flat, W)
    return out.reshape(x.shape + (D_MODEL,))


# speedup vs baseline: 1.0239x; 1.0239x over previous
import jax
import jax.numpy as jnp
from jax import lax
from jax.experimental import pallas as pl
from jax.experimental.pallas import tpu as pltpu, tpu_sc as plsc

D_MODEL = 64
NUM_CORES = 2
NUM_SUBCORES = 16
NUM_WORKERS = NUM_CORES * NUM_SUBCORES
LANES = 16

B_TOTAL = 1024 * 200
B_PER_W = B_TOTAL // NUM_WORKERS
CHUNK = 640
N_CHUNKS = B_PER_W // CHUNK
NBUF = 2

_SCALE = 8.0


def _emb_kernel(x_hbm, w_hbm, out_hbm, idx_v, rows0, rows1, g0, g1, o0, o1):
    wid = lax.axis_index("s") * NUM_CORES + lax.axis_index("c")
    base = wid * B_PER_W
    rows = (rows0, rows1)
    gsem = (g0, g1)
    osem = (o0, o1)

    pltpu.sync_copy(x_hbm.at[pl.ds(base, B_PER_W)], idx_v)

    def start_gather(c):
        pltpu.async_copy(
            w_hbm.at[idx_v.at[pl.ds(c * CHUNK, CHUNK)]],
            rows[c % NBUF],
            gsem[c % NBUF],
        )

    start_gather(0)
    for c in range(N_CHUNKS):
        b = c % NBUF
        if c + 1 < N_CHUNKS:
            nb = (c + 1) % NBUF
            if c >= 1:
                pltpu.make_async_copy(
                    rows[nb], out_hbm.at[pl.ds(base + (c - 1) * CHUNK, CHUNK)],
                    osem[nb],
                ).wait()
            start_gather(c + 1)
        pltpu.make_async_copy(
            w_hbm.at[idx_v.at[pl.ds(c * CHUNK, CHUNK)]], rows[b], gsem[b]
        ).wait()

        def scale_row(i, carry):
            for j in range(D_MODEL // LANES):
                sl = pl.ds(j * LANES, LANES)
                rows[b][i, sl] = rows[b][i, sl] * _SCALE
            return carry

        lax.fori_loop(0, CHUNK, scale_row, 0, unroll=4)
        pltpu.async_copy(
            rows[b], out_hbm.at[pl.ds(base + c * CHUNK, CHUNK)], osem[b]
        )

    for c in (N_CHUNKS - 2, N_CHUNKS - 1):
        pltpu.make_async_copy(
            rows[c % NBUF], out_hbm.at[pl.ds(base + c * CHUNK, CHUNK)],
            osem[c % NBUF],
        ).wait()


@jax.jit
def _emb(x_flat, w):
    mesh = plsc.VectorSubcoreMesh(core_axis_name="c", subcore_axis_name="s")
    run = pl.kernel(
        _emb_kernel,
        out_type=jax.ShapeDtypeStruct((B_TOTAL, D_MODEL), jnp.float32),
        mesh=mesh,
        scratch_types=[
            pltpu.VMEM((B_PER_W,), jnp.int32),
            pltpu.VMEM((CHUNK, D_MODEL), jnp.float32),
            pltpu.VMEM((CHUNK, D_MODEL), jnp.float32),
            pltpu.SemaphoreType.DMA,
            pltpu.SemaphoreType.DMA,
            pltpu.SemaphoreType.DMA,
            pltpu.SemaphoreType.DMA,
        ],
        compiler_params=pltpu.CompilerParams(use_tc_tiling_on_sc=False),
    )
    return run(x_flat, w)


def kernel(x, W):
    x_flat = x.reshape(-1).astype(jnp.int32)
    out = _emb(x_flat, W)
    return out.reshape(x.shape + (D_MODEL,))


# speedup vs baseline: 1.4920x; 1.4572x over previous
import jax
import jax.numpy as jnp
from jax import lax
from jax.experimental import pallas as pl
from jax.experimental.pallas import tpu as pltpu, tpu_sc as plsc

D_MODEL = 64
NUM_CORES = 2
NUM_SUBCORES = 16
NUM_WORKERS = NUM_CORES * NUM_SUBCORES
LANES = 16

BATCH = 1024
SEQ = 200
BATCH_PER_W = BATCH // NUM_WORKERS
ROWS_PER_W = BATCH_PER_W * SEQ
FULL_GROUPS = SEQ // LANES
TAIL = SEQ - FULL_GROUPS * LANES
IDX_BUF = ROWS_PER_W + 64

_SCALE = 8.0


def _emb_kernel(xf_hbm, w_hbm, out_hbm, idx_vm, r0, r1, g0, g1, o0, o1):
    wid = lax.axis_index("s") * NUM_CORES + lax.axis_index("c")
    base_b = wid * BATCH_PER_W
    rows = (r0, r1)
    gsem = (g0, g1)
    osem = (o0, o1)

    pltpu.sync_copy(
        xf_hbm.at[pl.ds(wid * ROWS_PER_W, ROWS_PER_W)],
        idx_vm.at[pl.ds(0, ROWS_PER_W)],
    )

    lane_iota = lax.iota(jnp.int32, LANES)

    def row_dma(vec, lane, dst_ref, dst_row, nb):
        idx = jnp.sum(jnp.where(lane_iota == lane, vec, 0))
        pltpu.async_copy(w_hbm.at[idx], dst_ref.at[dst_row], gsem[nb])

    def fire(c, nb):
        base = c * SEQ

        def group(g, carry):
            vec = idx_vm[pl.ds(base + g * LANES, LANES)]
            for l in range(LANES):
                row_dma(vec, l, rows[nb], g * LANES + l, nb)
            return carry

        lax.fori_loop(0, FULL_GROUPS, group, 0)
        vec = idx_vm[pl.ds(base + FULL_GROUPS * LANES, LANES)]
        for l in range(TAIL):
            row_dma(vec, l, rows[nb], FULL_GROUPS * LANES + l, nb)

    def drain_gather(nb):
        def body(i, carry):
            pltpu.make_async_copy(w_hbm.at[0], rows[nb].at[0], gsem[nb]).wait()
            return carry

        lax.fori_loop(0, SEQ, body, 0)

    def wait_out(c, nb):
        pltpu.make_async_copy(rows[nb], out_hbm.at[base_b + c], osem[nb]).wait()

    fire(0, 0)
    for c in range(BATCH_PER_W):
        nb = c % 2
        if c + 1 < BATCH_PER_W:
            nb2 = (c + 1) % 2
            if c >= 1:
                wait_out(c - 1, nb2)
            fire(c + 1, nb2)
        drain_gather(nb)

        def scale_row(i, carry):
            for j in range(D_MODEL // LANES):
                sl = pl.ds(j * LANES, LANES)
                rows[nb][i, sl] = rows[nb][i, sl] * _SCALE
            return carry

        lax.fori_loop(0, SEQ, scale_row, 0, unroll=4)
        pltpu.async_copy(rows[nb], out_hbm.at[base_b + c], osem[nb])

    wait_out(BATCH_PER_W - 2, (BATCH_PER_W - 2) % 2)
    wait_out(BATCH_PER_W - 1, (BATCH_PER_W - 1) % 2)


@jax.jit
def _emb(x_flat, w):
    mesh = plsc.VectorSubcoreMesh(core_axis_name="c", subcore_axis_name="s")
    run = pl.kernel(
        _emb_kernel,
        out_type=jax.ShapeDtypeStruct((BATCH, SEQ, D_MODEL), jnp.float32),
        mesh=mesh,
        scratch_types=[
            pltpu.VMEM((IDX_BUF,), jnp.int32),
            pltpu.VMEM((SEQ, D_MODEL), jnp.float32),
            pltpu.VMEM((SEQ, D_MODEL), jnp.float32),
            pltpu.SemaphoreType.DMA,
            pltpu.SemaphoreType.DMA,
            pltpu.SemaphoreType.DMA,
            pltpu.SemaphoreType.DMA,
        ],
        compiler_params=pltpu.CompilerParams(needs_layout_passes=False),
    )
    return run(x_flat, w)


def kernel(x, W):
    x_flat = x.reshape(-1).astype(jnp.int32)
    return _emb(x_flat, W)


# speedup vs baseline: 1.4943x; 1.0016x over previous
import jax
import jax.numpy as jnp
from jax import lax
from jax.experimental import pallas as pl
from jax.experimental.pallas import tpu as pltpu, tpu_sc as plsc

D_MODEL = 64
NUM_CORES = 2
NUM_SUBCORES = 16
NUM_WORKERS = NUM_CORES * NUM_SUBCORES
LANES = 16

BATCH = 1024
SEQ = 200
BATCH_PER_W = BATCH // NUM_WORKERS
ROWS_PER_W = BATCH_PER_W * SEQ
FULL_GROUPS = SEQ // LANES
TAIL = SEQ - FULL_GROUPS * LANES
IDX_BUF = ROWS_PER_W + 64

_SCALE = 8.0


def _emb_kernel(xf_hbm, w_hbm, out_hbm, idx_vm, r0, r1, g0, g1, o0, o1):
    wid = lax.axis_index("s") * NUM_CORES + lax.axis_index("c")
    base_b = wid * BATCH_PER_W
    rows = (r0, r1)
    gsem = (g0, g1)
    osem = (o0, o1)

    pltpu.sync_copy(
        xf_hbm.at[pl.ds(wid * ROWS_PER_W, ROWS_PER_W)],
        idx_vm.at[pl.ds(0, ROWS_PER_W)],
    )

    lane_iota = lax.iota(jnp.int32, LANES)

    def row_dma(vec, lane, dst_ref, dst_row, nb):
        idx = jnp.sum(jnp.where(lane_iota == lane, vec, 0))
        pltpu.async_copy(w_hbm.at[idx], dst_ref.at[dst_row], gsem[nb])

    def fire(c, nb):
        base = c * SEQ

        def group(g, carry):
            vec = idx_vm[pl.ds(base + g * LANES, LANES)]
            for l in range(LANES):
                row_dma(vec, l, rows[nb], g * LANES + l, nb)
            return carry

        lax.fori_loop(0, FULL_GROUPS, group, 0)
        vec = idx_vm[pl.ds(base + FULL_GROUPS * LANES, LANES)]
        for l in range(TAIL):
            row_dma(vec, l, rows[nb], FULL_GROUPS * LANES + l, nb)

    def drain_gather(nb):
        def body(i, carry):
            pltpu.make_async_copy(w_hbm.at[0], rows[nb].at[0], gsem[nb]).wait()
            return carry

        lax.fori_loop(0, SEQ, body, 0)

    def wait_out(c, nb):
        pltpu.make_async_copy(rows[nb], out_hbm.at[base_b + c], osem[nb]).wait()

    fire(0, 0)
    for c in range(BATCH_PER_W):
        nb = c % 2
        if c + 1 < BATCH_PER_W:
            nb2 = (c + 1) % 2
            if c >= 1:
                wait_out(c - 1, nb2)
            fire(c + 1, nb2)
        drain_gather(nb)

        def scale_row(i, carry):
            for j in range(D_MODEL // LANES):
                sl = pl.ds(j * LANES, LANES)
                rows[nb][i, sl] = rows[nb][i, sl] * _SCALE
            return carry

        lax.fori_loop(0, SEQ, scale_row, 0, unroll=4)
        pltpu.async_copy(rows[nb], out_hbm.at[base_b + c], osem[nb])

    wait_out(BATCH_PER_W - 2, (BATCH_PER_W - 2) % 2)
    wait_out(BATCH_PER_W - 1, (BATCH_PER_W - 1) % 2)


@jax.jit
def _emb(x_flat, w):
    mesh = plsc.VectorSubcoreMesh(core_axis_name="c", subcore_axis_name="s")
    run = pl.kernel(
        _emb_kernel,
        out_type=jax.ShapeDtypeStruct((BATCH, SEQ, D_MODEL), jnp.float32),
        mesh=mesh,
        scratch_types=[
            pltpu.VMEM((IDX_BUF,), jnp.int32),
            pltpu.VMEM((SEQ, D_MODEL), jnp.float32),
            pltpu.VMEM((SEQ, D_MODEL), jnp.float32),
            pltpu.SemaphoreType.DMA,
            pltpu.SemaphoreType.DMA,
            pltpu.SemaphoreType.DMA,
            pltpu.SemaphoreType.DMA,
        ],
        compiler_params=pltpu.CompilerParams(
            needs_layout_passes=False, use_tc_tiling_on_sc=True
        ),
    )
    return run(x_flat, w)


def kernel(x, W):
    x_flat = x.reshape(-1).astype(jnp.int32)
    return _emb(x_flat, W)
